# baseline (device time: 19661 ns/iter reference)
import jax
import jax.numpy as jnp
from jax import lax
from jax.experimental import pallas as pl
from jax.experimental.pallas import tpu as pltpu

N_DEV = 4
N_LAYERS = 3
NB = 2


def kernel(x, Win0, Wout0, Win1, Wout1, Win2, Wout2):
    b, d = x.shape
    hid = Win0.shape[1]
    rb = b // NB

    def body(
        x_hbm,
        win0_hbm,
        wout0_hbm,
        win1_hbm,
        wout1_hbm,
        win2_hbm,
        wout2_hbm,
        out_hbm,
        x_v,
        win_v,
        wout_v,
        out_v,
        send_buf,
        recv_buf,
        load_sems,
        send_sems,
        recv_sems,
    ):
        my = lax.axis_index("i")

        loads = []
        srcs = [x_hbm, win0_hbm, wout0_hbm, win1_hbm, wout1_hbm, win2_hbm, wout2_hbm]
        dsts = [x_v, win_v.at[0], wout_v.at[0], win_v.at[1], wout_v.at[1],
                win_v.at[2], wout_v.at[2]]
        for i, (s, dst) in enumerate(zip(srcs, dsts)):
            cp = pltpu.make_async_copy(s, dst, load_sems.at[i])
            cp.start()
            loads.append(cp)

        barrier_sem = pltpu.get_barrier_semaphore()
        for off in (1, 2, 3):
            pl.semaphore_signal(
                barrier_sem,
                inc=1,
                device_id=((my + off) % N_DEV,),
                device_id_type=pl.DeviceIdType.MESH,
            )
        pl.semaphore_wait(barrier_sem, N_DEV - 1)

        all_sends = []

        def send_block(l, blk, p_bf16):
            send_buf[l, blk, :, :] = p_bf16
            for j in (1, 0, 2):
                peer = (my + j + 1) % N_DEV
                rdma = pltpu.make_async_remote_copy(
                    src_ref=send_buf.at[l, blk],
                    dst_ref=recv_buf.at[l, blk, 2 - j],
                    send_sem=send_sems.at[l, blk, j],
                    recv_sem=recv_sems.at[l, blk, 2 - j],
                    device_id=(peer,),
                    device_id_type=pl.DeviceIdType.MESH,
                )
                rdma.start()
                all_sends.append(rdma)

        def gather_block(l, blk, p_local):
            acc = p_local
            for s in (0, 2, 1):
                recv = pltpu.make_async_remote_copy(
                    src_ref=send_buf.at[l, blk],
                    dst_ref=recv_buf.at[l, blk, s],
                    send_sem=send_sems.at[l, blk, s],
                    recv_sem=recv_sems.at[l, blk, s],
                    device_id=(my,),
                    device_id_type=pl.DeviceIdType.MESH,
                )
                recv.wait_recv()
                acc = acc + recv_buf[l, blk, s, :, :].astype(jnp.float32)
            return acc

        def layer(l, x_blk):
            h = jnp.maximum(
                jnp.dot(x_blk, win_v[l, :, :], preferred_element_type=jnp.float32),
                0.0,
            )
            return jnp.dot(h, wout_v[l, :, :], preferred_element_type=jnp.float32)

        loads[0].wait()
        loads[1].wait()
        loads[2].wait()
        p_blocks = []
        for blk in range(NB):
            p = layer(0, x_v[pl.ds(blk * rb, rb), :])
            send_block(0, blk, p.astype(jnp.bfloat16))
            p_blocks.append(p)

        for l in range(1, N_LAYERS):
            loads[2 * l + 1].wait()
            loads[2 * l + 2].wait()
            for blk in range(NB):
                x_blk = gather_block(l - 1, blk, p_blocks[blk])
                p = layer(l, x_blk)
                send_block(l, blk, p.astype(jnp.bfloat16))
                p_blocks[blk] = p

        for blk in range(NB):
            out_v[pl.ds(blk * rb, rb), :] = gather_block(
                N_LAYERS - 1, blk, p_blocks[blk]
            )
        out_cp = pltpu.make_async_copy(out_v, out_hbm, load_sems.at[7])
        out_cp.start()

        for rdma in all_sends:
            rdma.wait_send()
        out_cp.wait()

    return pl.pallas_call(
        body,
        out_shape=jax.ShapeDtypeStruct((b, d), jnp.float32),
        in_specs=[pl.BlockSpec(memory_space=pl.ANY)] * 7,
        out_specs=pl.BlockSpec(memory_space=pl.ANY),
        scratch_shapes=[
            pltpu.VMEM((b, d), jnp.float32),
            pltpu.VMEM((N_LAYERS, d, hid), jnp.float32),
            pltpu.VMEM((N_LAYERS, hid, d), jnp.float32),
            pltpu.VMEM((b, d), jnp.float32),
            pltpu.VMEM((N_LAYERS, NB, rb, d), jnp.bfloat16),
            pltpu.VMEM((N_LAYERS, NB, 3, rb, d), jnp.bfloat16),
            pltpu.SemaphoreType.DMA((8,)),
            pltpu.SemaphoreType.DMA((N_LAYERS, NB, 3)),
            pltpu.SemaphoreType.DMA((N_LAYERS, NB, 3)),
        ],
        compiler_params=pltpu.CompilerParams(collective_id=0),
    )(x, Win0, Wout0, Win1, Wout1, Win2, Wout2)


# device time: 13262 ns/iter; 1.4825x vs baseline; 1.4825x over previous
import jax
import jax.numpy as jnp
from jax import lax
from jax.experimental import pallas as pl
from jax.experimental.pallas import tpu as pltpu

N_DEV = 4
N_LAYERS = 3
NB = 2


def kernel(x, Win0, Wout0, Win1, Wout1, Win2, Wout2):
    b, d = x.shape
    hid = Win0.shape[1]
    rb = b // NB

    wincat = jnp.concatenate([Win0, Win1, Win2], axis=0)
    xwcat = jnp.concatenate([x, Wout0, Wout1, Wout2], axis=0)

    def body(
        wincat_ref,
        xwcat_ref,
        out_ref,
        send_buf,
        recv_buf,
        send_sems,
        recv_sems,
    ):
        my = lax.axis_index("i")

        barrier_sem = pltpu.get_barrier_semaphore()
        for off in (1, 2, 3):
            pl.semaphore_signal(
                barrier_sem,
                inc=1,
                device_id=((my + off) % N_DEV,),
                device_id_type=pl.DeviceIdType.MESH,
            )
        pl.semaphore_wait(barrier_sem, N_DEV - 1)

        all_sends = []

        def send_block(l, blk, p_bf16):
            send_buf[l, blk, :, :] = p_bf16
            for j in (1, 0, 2):
                peer = (my + j + 1) % N_DEV
                rdma = pltpu.make_async_remote_copy(
                    src_ref=send_buf.at[l, blk],
                    dst_ref=recv_buf.at[l, blk, 2 - j],
                    send_sem=send_sems.at[l, blk, j],
                    recv_sem=recv_sems.at[l, blk, 2 - j],
                    device_id=(peer,),
                    device_id_type=pl.DeviceIdType.MESH,
                )
                rdma.start()
                all_sends.append(rdma)

        def gather_block(l, blk, p_local):
            acc = p_local
            for s in (0, 2, 1):
                recv = pltpu.make_async_remote_copy(
                    src_ref=send_buf.at[l, blk],
                    dst_ref=recv_buf.at[l, blk, s],
                    send_sem=send_sems.at[l, blk, s],
                    recv_sem=recv_sems.at[l, blk, s],
                    device_id=(my,),
                    device_id_type=pl.DeviceIdType.MESH,
                )
                recv.wait_recv()
                acc = acc + recv_buf[l, blk, s, :, :].astype(jnp.float32)
            return acc

        def layer(l, x_blk):
            win = wincat_ref[pl.ds(l * d, d), :]
            wout = xwcat_ref[pl.ds(b + l * hid, hid), :]
            h = jnp.maximum(
                jnp.dot(x_blk, win, preferred_element_type=jnp.float32),
                0.0,
            )
            return jnp.dot(h, wout, preferred_element_type=jnp.float32)

        p_blocks = []
        for blk in range(NB):
            p = layer(0, xwcat_ref[pl.ds(blk * rb, rb), :])
            send_block(0, blk, p.astype(jnp.bfloat16))
            p_blocks.append(p)

        for l in range(1, N_LAYERS):
            for blk in range(NB):
                x_blk = gather_block(l - 1, blk, p_blocks[blk])
                p = layer(l, x_blk)
                send_block(l, blk, p.astype(jnp.bfloat16))
                p_blocks[blk] = p

        for blk in range(NB):
            out_ref[pl.ds(blk * rb, rb), :] = gather_block(
                N_LAYERS - 1, blk, p_blocks[blk]
            )

        for rdma in all_sends:
            rdma.wait_send()

    return pl.pallas_call(
        body,
        out_shape=jax.ShapeDtypeStruct((b, d), jnp.float32),
        in_specs=[pl.BlockSpec(memory_space=pltpu.VMEM)] * 2,
        out_specs=pl.BlockSpec(memory_space=pltpu.VMEM),
        scratch_shapes=[
            pltpu.VMEM((N_LAYERS, NB, rb, d), jnp.bfloat16),
            pltpu.VMEM((N_LAYERS, NB, 3, rb, d), jnp.bfloat16),
            pltpu.SemaphoreType.DMA((N_LAYERS, NB, 3)),
            pltpu.SemaphoreType.DMA((N_LAYERS, NB, 3)),
        ],
        compiler_params=pltpu.CompilerParams(collective_id=0),
    )(wincat, xwcat)
